# Initial kernel scaffold; baseline (speedup 1.0000x reference)
import jax
import jax.numpy as jnp
from jax.experimental import pallas as pl


def kernel(point_set_1, point_set_2):
    raise NotImplementedError("write your pallas kernel here")


# speedup vs baseline: 1.7558x; 1.7558x over previous
import jax
import jax.numpy as jnp
from jax.experimental import pallas as pl


def _chamfer_body(x1_ref, y1_ref, x2_ref, y2_ref, out_ref):
    x1 = x1_ref[0, 0, :]
    y1 = y1_ref[0, 0, :]
    x2 = x2_ref[0, 0, :]
    y2 = y2_ref[0, 0, :]
    dx = x1[:, None] - x2[None, :]
    dy = y1[:, None] - y2[None, :]
    d2 = dx * dx + dy * dy
    rmin = jnp.min(d2, axis=1)
    cmin = jnp.min(d2, axis=0)
    d_fwd = jnp.mean(jnp.sqrt(rmin))
    d_bwd = jnp.mean(jnp.sqrt(cmin))
    out_ref[...] = ((d_fwd + d_bwd) * 0.5).reshape(1, 1, 1)


def kernel(point_set_1, point_set_2):
    b, p1, _ = point_set_1.shape
    p2 = point_set_2.shape[1]
    x1 = point_set_1[:, :, 0].reshape(b, 1, p1)
    y1 = point_set_1[:, :, 1].reshape(b, 1, p1)
    x2 = point_set_2[:, :, 0].reshape(b, 1, p2)
    y2 = point_set_2[:, :, 1].reshape(b, 1, p2)
    out = pl.pallas_call(
        _chamfer_body,
        grid=(b,),
        in_specs=[
            pl.BlockSpec((1, 1, p1), lambda i: (i, 0, 0)),
            pl.BlockSpec((1, 1, p1), lambda i: (i, 0, 0)),
            pl.BlockSpec((1, 1, p2), lambda i: (i, 0, 0)),
            pl.BlockSpec((1, 1, p2), lambda i: (i, 0, 0)),
        ],
        out_specs=pl.BlockSpec((1, 1, 1), lambda i: (i, 0, 0)),
        out_shape=jax.ShapeDtypeStruct((b, 1, 1), jnp.float32),
    )(x1, y1, x2, y2)
    return out[:, 0, 0]
